# baseline (speedup 1.0000x reference)
import functools

import jax
import jax.numpy as jnp
from jax import lax
from jax.experimental import pallas as pl
from jax.experimental.pallas import tpu as pltpu
from jax.experimental.pallas import tpu_sc as plsc

N = 10000
E = 320000
F = 16
CH = 1000
ROWS = E // CH
NC, NS = 2, 16
NW = NC * NS
RPW = ROWS // NW
NPT = N // NS

assert ROWS * CH == E and RPW * NW == ROWS and NPT * NS == N

_MESH = plsc.VectorSubcoreMesh(core_axis_name="c", subcore_axis_name="s")


@functools.partial(
    pl.kernel,
    out_type=(jax.ShapeDtypeStruct((N,), jnp.float32),
              jax.ShapeDtypeStruct((N,), jnp.float32)),
    mesh=_MESH,
    scratch_types=[
        pltpu.VMEM((RPW, CH), jnp.int32),
        pltpu.VMEM((CH,), jnp.float32),
        pltpu.VMEM_SHARED((N,), jnp.float32),
        pltpu.SemaphoreType.DMA,
    ],
    compiler_params=pltpu.CompilerParams(use_tc_tiling_on_sc=False),
)
def _deg_kernel(dst_hbm, ones_hbm, zeros_hbm, out0_hbm, out1_hbm,
                dst_v, ones_v, acc, sem):
    c = lax.axis_index("c")
    s = lax.axis_index("s")
    wid = s * NC + c
    pltpu.sync_copy(dst_hbm.at[pl.ds(wid * RPW, RPW)], dst_v)
    pltpu.sync_copy(ones_hbm, ones_v)

    @pl.when(s == 0)
    def _():
        pltpu.sync_copy(zeros_hbm, acc)

    plsc.subcore_barrier()

    descs = [pltpu.async_copy(ones_v, acc.at[dst_v.at[j]], sem, add=True)
             for j in range(RPW)]
    for d in descs:
        d.wait()
    plsc.subcore_barrier()

    @pl.when((s == 0) & (c == 0))
    def _():
        pltpu.sync_copy(acc, out0_hbm)

    @pl.when((s == 0) & (c == 1))
    def _():
        pltpu.sync_copy(acc, out1_hbm)


def _make_prop_kernel(Fw):
    @functools.partial(
        pl.kernel,
        out_type=jax.ShapeDtypeStruct((NC, N, Fw), jnp.float32),
        mesh=_MESH,
        scratch_types=[
            pltpu.VMEM((RPW, CH), jnp.int32),
            pltpu.VMEM((RPW, CH), jnp.int32),
            pltpu.VMEM((4, CH, Fw), jnp.float32),
            pltpu.VMEM_SHARED((N, Fw), jnp.float32),
            [pltpu.SemaphoreType.DMA] * 4,
            [pltpu.SemaphoreType.DMA] * 4,
        ],
        compiler_params=pltpu.CompilerParams(use_tc_tiling_on_sc=False),
    )
    def prop(src_hbm, dst_hbm, g_hbm, zeros_hbm, out_hbm,
             src_v, dst_v, buf, acc, gsems, ssems):
        c = lax.axis_index("c")
        s = lax.axis_index("s")
        wid = s * NC + c
        NB = 4
        pltpu.sync_copy(src_hbm.at[pl.ds(wid * RPW, RPW)], src_v)
        pltpu.sync_copy(dst_hbm.at[pl.ds(wid * RPW, RPW)], dst_v)

        pltpu.sync_copy(zeros_hbm.at[pl.ds(s * NPT, NPT)],
                        acc.at[pl.ds(s * NPT, NPT)])

        def fire_gather(j):
            return pltpu.async_copy(g_hbm.at[src_v.at[j]], buf.at[j % NB],
                                    gsems[j % NB])

        def fire_scatter(j):
            return pltpu.async_copy(buf.at[j % NB], acc.at[dst_v.at[j]],
                                    ssems[j % NB], add=True)

        gd = {0: fire_gather(0)}
        if RPW > 1:
            gd[1] = fire_gather(1)
        plsc.subcore_barrier()
        sd = {}
        waited = set()
        for j in range(RPW):
            gd[j].wait()
            sd[j] = fire_scatter(j)
            jj = j + 2
            if jj < RPW:
                if jj - NB >= 0:
                    sd[jj - NB].wait()
                    waited.add(jj - NB)
                gd[jj] = fire_gather(jj)
        for j in range(RPW):
            if j not in waited:
                sd[j].wait()
        plsc.subcore_barrier()
        pltpu.sync_copy(acc.at[pl.ds(s * NPT, NPT)],
                        out_hbm.at[c].at[pl.ds(s * NPT, NPT)])

    return prop


_prop_kernel = _make_prop_kernel(F)
_prop_kernel4 = _make_prop_kernel(4)


NP8 = N // 8


def _tc_a_body(d0_ref, d1_ref, s_ref, s4_ref, x_ref, w1b_ref,
               dis_ref, dis4_ref, g_ref):
    d = d0_ref[...] + d1_ref[...] + 1.0
    dis2 = lax.rsqrt(d)
    disP = jnp.dot(dis2, s_ref[...],
                   preferred_element_type=jnp.float32)
    dis4_ref[...] = jnp.dot(dis2, s4_ref[...],
                            preferred_element_type=jnp.float32)
    hP = jnp.dot(x_ref[:, 0, :], w1b_ref[0],
                 preferred_element_type=jnp.float32)
    for b in range(1, 8):
        hP = hP + jnp.dot(x_ref[:, b, :], w1b_ref[b],
                          preferred_element_type=jnp.float32)
    dis_ref[...] = disP
    g_ref[...] = hP * disP


def _tc_b_body(p_ref, g1_ref, dis_ref, dis4_ref, b1p_ref, w2p_ref, q_ref):
    disP = dis_ref[...]
    z = (p_ref[0] + p_ref[1] + g1_ref[...]) * disP + b1p_ref[...][None, :]
    h1p = jnp.maximum(z, 0.0)
    q_ref[...] = jnp.dot(h1p, w2p_ref[...],
                         preferred_element_type=jnp.float32) * dis4_ref[...]


def _tc_c_body(p_ref, q_ref, dis4_ref, b2p_ref, gsum_ref, out_ref):
    h2 = (p_ref[0] + p_ref[1] + q_ref[...]) * dis4_ref[...]
    h2 = h2 + b2p_ref[...][None, :]
    m = jnp.max(h2, axis=1, keepdims=True)
    e = jnp.exp(h2 - m)
    gs = jnp.dot(e, gsum_ref[...],
                 preferred_element_type=jnp.float32)
    out_ref[...] = h2 - m - jnp.log(gs)


_tc_a = pl.pallas_call(
    _tc_a_body,
    out_shape=(jax.ShapeDtypeStruct((NP8, 128), jnp.float32),
               jax.ShapeDtypeStruct((NP8, 32), jnp.float32),
               jax.ShapeDtypeStruct((NP8, 128), jnp.float32)))
_tc_b = pl.pallas_call(
    _tc_b_body, out_shape=jax.ShapeDtypeStruct((NP8, 32), jnp.float32))
_tc_c = pl.pallas_call(
    _tc_c_body, out_shape=jax.ShapeDtypeStruct((NP8, 32), jnp.float32))


def kernel(x, edge_index, W1, b1, W2, b2):
    ei = edge_index.astype(jnp.int32)
    src2 = ei[0].reshape(ROWS, CH)
    dst2 = ei[1].reshape(ROWS, CH)
    ones_ch = jnp.ones((CH,), jnp.float32)
    zeros_n = jnp.zeros((N,), jnp.float32)
    zeros_nf = jnp.zeros((N, F), jnp.float32)
    zeros_n4 = jnp.zeros((N, 4), jnp.float32)

    S = jnp.repeat(jnp.eye(8, dtype=jnp.float32), F, axis=1)
    S4 = jnp.repeat(jnp.eye(8, dtype=jnp.float32), 4, axis=1)
    P = jax.nn.one_hot(
        F * jnp.arange(8)[:, None] + jnp.arange(F)[None, :], 128,
        dtype=jnp.float32)
    W1B = jnp.einsum('cf,bfk->bck', W1, P)
    Q = jax.nn.one_hot(
        4 * jnp.arange(8)[:, None] + jnp.arange(4)[None, :], 32,
        dtype=jnp.float32)
    W2P = jnp.einsum('bck,cf,bfj->kj', P, W2, Q)
    b1P = jnp.tile(b1, 8)
    b2P = jnp.tile(b2, 8)
    G = jnp.kron(jnp.eye(8, dtype=jnp.float32),
                 jnp.ones((4, 4), jnp.float32))
    xp = x.reshape(NP8, 8, 128)

    deg0, deg1 = _deg_kernel(dst2, ones_ch, zeros_n)
    d0 = deg0.reshape(NP8, 8)
    d1 = deg1.reshape(NP8, 8)
    disP, dis4P, g1P = _tc_a(d0, d1, S, S4, xp, W1B)
    p1 = _prop_kernel(src2, dst2, g1P.reshape(N, F), zeros_nf)
    qP = _tc_b(p1.reshape(2, NP8, 128), g1P, disP, dis4P, b1P, W2P)
    p2 = _prop_kernel4(src2, dst2, qP.reshape(N, 4), zeros_n4)
    outP = _tc_c(p2.reshape(2, NP8, 32), qP, dis4P, b2P, G)
    return outP.reshape(N, 4)


# speedup vs baseline: 1.2757x; 1.2757x over previous
import functools

import jax
import jax.numpy as jnp
from jax import lax
from jax.experimental import pallas as pl
from jax.experimental.pallas import tpu as pltpu
from jax.experimental.pallas import tpu_sc as plsc

N = 10000
E = 320000
F = 16
CH = 1000
ROWS = E // CH
NC, NS = 2, 16
NW = NC * NS
RPW = ROWS // NW
NPT = N // NS

assert ROWS * CH == E and RPW * NW == ROWS and NPT * NS == N

_MESH = plsc.VectorSubcoreMesh(core_axis_name="c", subcore_axis_name="s")


@functools.partial(
    pl.kernel,
    out_type=(jax.ShapeDtypeStruct((N,), jnp.float32),
              jax.ShapeDtypeStruct((N,), jnp.float32)),
    mesh=_MESH,
    scratch_types=[
        pltpu.VMEM((RPW, CH), jnp.int32),
        pltpu.VMEM((CH,), jnp.float32),
        pltpu.VMEM_SHARED((N,), jnp.float32),
        pltpu.SemaphoreType.DMA,
    ],
    compiler_params=pltpu.CompilerParams(use_tc_tiling_on_sc=False),
)
def _deg_kernel(dst_hbm, ones_hbm, zeros_hbm, out0_hbm, out1_hbm,
                dst_v, ones_v, acc, sem):
    c = lax.axis_index("c")
    s = lax.axis_index("s")
    wid = s * NC + c
    pltpu.sync_copy(dst_hbm.at[pl.ds(wid * RPW, RPW)], dst_v)
    pltpu.sync_copy(ones_hbm, ones_v)

    @pl.when(s == 0)
    def _():
        pltpu.sync_copy(zeros_hbm, acc)

    plsc.subcore_barrier()

    descs = [pltpu.async_copy(ones_v, acc.at[dst_v.at[j]], sem, add=True)
             for j in range(RPW)]
    for d in descs:
        d.wait()
    plsc.subcore_barrier()

    @pl.when((s == 0) & (c == 0))
    def _():
        pltpu.sync_copy(acc, out0_hbm)

    @pl.when((s == 0) & (c == 1))
    def _():
        pltpu.sync_copy(acc, out1_hbm)


def _make_prop_kernel(Fw):
    @functools.partial(
        pl.kernel,
        out_type=jax.ShapeDtypeStruct((NC, N, Fw), jnp.float32),
        mesh=_MESH,
        scratch_types=[
            pltpu.VMEM((RPW, CH), jnp.int32),
            pltpu.VMEM((RPW, CH), jnp.int32),
            pltpu.VMEM((4, CH, Fw), jnp.float32),
            pltpu.VMEM_SHARED((N, Fw), jnp.float32),
            [pltpu.SemaphoreType.DMA] * 4,
            [pltpu.SemaphoreType.DMA] * 4,
        ],
        compiler_params=pltpu.CompilerParams(use_tc_tiling_on_sc=False),
    )
    def prop(src_hbm, dst_hbm, g_hbm, zeros_hbm, out_hbm,
             src_v, dst_v, buf, acc, gsems, ssems):
        c = lax.axis_index("c")
        s = lax.axis_index("s")
        wid = s * NC + c
        NB = 4
        pltpu.sync_copy(src_hbm.at[pl.ds(wid * RPW, RPW)], src_v)
        pltpu.sync_copy(dst_hbm.at[pl.ds(wid * RPW, RPW)], dst_v)

        pltpu.sync_copy(zeros_hbm.at[pl.ds(s * NPT, NPT)],
                        acc.at[pl.ds(s * NPT, NPT)])

        def fire_gather(j):
            return pltpu.async_copy(g_hbm.at[src_v.at[j]], buf.at[j % NB],
                                    gsems[j % NB])

        def fire_scatter(j):
            return pltpu.async_copy(buf.at[j % NB], acc.at[dst_v.at[j]],
                                    ssems[j % NB], add=True)

        gd = {0: fire_gather(0)}
        if RPW > 1:
            gd[1] = fire_gather(1)
        plsc.subcore_barrier()
        sd = {}
        waited = set()
        for j in range(RPW):
            gd[j].wait()
            sd[j] = fire_scatter(j)
            jj = j + 2
            if jj < RPW:
                if jj - NB >= 0:
                    sd[jj - NB].wait()
                    waited.add(jj - NB)
                gd[jj] = fire_gather(jj)
        for j in range(RPW):
            if j not in waited:
                sd[j].wait()
        plsc.subcore_barrier()
        pltpu.sync_copy(acc.at[pl.ds(s * NPT, NPT)],
                        out_hbm.at[c].at[pl.ds(s * NPT, NPT)])

    return prop


_prop_kernel = _make_prop_kernel(F)


NP8 = N // 8


def _tc_a_body(d0_ref, d1_ref, s_ref, x_ref, w1b_ref, dis_ref, g_ref):
    d = d0_ref[...] + d1_ref[...] + 1.0
    dis2 = lax.rsqrt(d)
    disP = jnp.dot(dis2, s_ref[...],
                   preferred_element_type=jnp.float32)
    hP = jnp.dot(x_ref[:, 0, :], w1b_ref[0],
                 preferred_element_type=jnp.float32)
    for b in range(1, 8):
        hP = hP + jnp.dot(x_ref[:, b, :], w1b_ref[b],
                          preferred_element_type=jnp.float32)
    dis_ref[...] = disP
    g_ref[...] = hP * disP


def _tc_b_body(p_ref, g1_ref, dis_ref, b1p_ref, g2_ref):
    disP = dis_ref[...]
    z = (p_ref[0] + p_ref[1] + g1_ref[...]) * disP + b1p_ref[...][None, :]
    g2_ref[...] = jnp.maximum(z, 0.0) * disP


def _tc_c_body(p_ref, g2_ref, dis_ref, w2p_ref, b2p_ref, gsum_ref, out_ref):
    zP = (p_ref[0] + p_ref[1] + g2_ref[...]) * dis_ref[...]
    h2 = jnp.dot(zP, w2p_ref[...],
                 preferred_element_type=jnp.float32)
    h2 = h2 + b2p_ref[...][None, :]
    m = jnp.max(h2, axis=1, keepdims=True)
    e = jnp.exp(h2 - m)
    gs = jnp.dot(e, gsum_ref[...],
                 preferred_element_type=jnp.float32)
    out_ref[...] = h2 - m - jnp.log(gs)


_tc_a = pl.pallas_call(
    _tc_a_body,
    out_shape=(jax.ShapeDtypeStruct((NP8, 128), jnp.float32),
               jax.ShapeDtypeStruct((NP8, 128), jnp.float32)))
_tc_b = pl.pallas_call(
    _tc_b_body, out_shape=jax.ShapeDtypeStruct((NP8, 128), jnp.float32))
_tc_c = pl.pallas_call(
    _tc_c_body, out_shape=jax.ShapeDtypeStruct((NP8, 32), jnp.float32))


def kernel(x, edge_index, W1, b1, W2, b2):
    ei = edge_index.astype(jnp.int32)
    src2 = ei[0].reshape(ROWS, CH)
    dst2 = ei[1].reshape(ROWS, CH)
    ones_ch = jnp.ones((CH,), jnp.float32)
    zeros_n = jnp.zeros((N,), jnp.float32)
    zeros_nf = jnp.zeros((N, F), jnp.float32)

    S = jnp.repeat(jnp.eye(8, dtype=jnp.float32), F, axis=1)
    P = jax.nn.one_hot(
        F * jnp.arange(8)[:, None] + jnp.arange(F)[None, :], 128,
        dtype=jnp.float32)
    W1B = jnp.einsum('cf,bfk->bck', W1, P)
    Q = jax.nn.one_hot(
        4 * jnp.arange(8)[:, None] + jnp.arange(4)[None, :], 32,
        dtype=jnp.float32)
    W2P = jnp.einsum('bck,cf,bfj->kj', P, W2, Q)
    b1P = jnp.tile(b1, 8)
    b2P = jnp.tile(b2, 8)
    G = jnp.kron(jnp.eye(8, dtype=jnp.float32),
                 jnp.ones((4, 4), jnp.float32))
    xp = x.reshape(NP8, 8, 128)

    deg0, deg1 = _deg_kernel(dst2, ones_ch, zeros_n)
    d0 = deg0.reshape(NP8, 8)
    d1 = deg1.reshape(NP8, 8)
    disP, g1P = _tc_a(d0, d1, S, xp, W1B)
    p1 = _prop_kernel(src2, dst2, g1P.reshape(N, F), zeros_nf)
    g2P = _tc_b(p1.reshape(2, NP8, 128), g1P, disP, b1P)
    p2 = _prop_kernel(src2, dst2, g2P.reshape(N, F), zeros_nf)
    outP = _tc_c(p2.reshape(2, NP8, 128), g2P, disP, W2P, b2P, G)
    return outP.reshape(N, 4)


# speedup vs baseline: 1.2829x; 1.0057x over previous
import functools

import jax
import jax.numpy as jnp
from jax import lax
from jax.experimental import pallas as pl
from jax.experimental.pallas import tpu as pltpu
from jax.experimental.pallas import tpu_sc as plsc

N = 10000
E = 320000
F = 16
CH = 1000
ROWS = E // CH
NC, NS = 2, 16
NW = NC * NS
RPW = ROWS // NW
NPT = N // NS

assert ROWS * CH == E and RPW * NW == ROWS and NPT * NS == N

_MESH = plsc.VectorSubcoreMesh(core_axis_name="c", subcore_axis_name="s")


@functools.partial(
    pl.kernel,
    out_type=(jax.ShapeDtypeStruct((N,), jnp.float32),
              jax.ShapeDtypeStruct((N,), jnp.float32)),
    mesh=_MESH,
    scratch_types=[
        pltpu.VMEM((RPW, CH), jnp.int32),
        pltpu.VMEM((CH,), jnp.float32),
        pltpu.VMEM_SHARED((N,), jnp.float32),
        pltpu.SemaphoreType.DMA,
    ],
    compiler_params=pltpu.CompilerParams(use_tc_tiling_on_sc=False),
)
def _deg_kernel(dst_hbm, ones_hbm, zeros_hbm, out0_hbm, out1_hbm,
                dst_v, ones_v, acc, sem):
    c = lax.axis_index("c")
    s = lax.axis_index("s")
    wid = s * NC + c
    pltpu.sync_copy(dst_hbm.at[pl.ds(wid * RPW, RPW)], dst_v)
    pltpu.sync_copy(ones_hbm, ones_v)

    @pl.when(s == 0)
    def _():
        pltpu.sync_copy(zeros_hbm, acc)

    plsc.subcore_barrier()

    descs = [pltpu.async_copy(ones_v, acc.at[dst_v.at[j]], sem, add=True)
             for j in range(RPW)]
    for d in descs:
        d.wait()
    plsc.subcore_barrier()

    @pl.when((s == 0) & (c == 0))
    def _():
        pltpu.sync_copy(acc, out0_hbm)

    @pl.when((s == 0) & (c == 1))
    def _():
        pltpu.sync_copy(acc, out1_hbm)


def _make_prop_kernel(Fw):
    @functools.partial(
        pl.kernel,
        out_type=jax.ShapeDtypeStruct((NC, N, Fw), jnp.float32),
        mesh=_MESH,
        scratch_types=[
            pltpu.VMEM((RPW, CH), jnp.int32),
            pltpu.VMEM((RPW, CH), jnp.int32),
            pltpu.VMEM((4, CH, Fw), jnp.float32),
            pltpu.VMEM_SHARED((N, Fw), jnp.float32),
            [pltpu.SemaphoreType.DMA] * 4,
            [pltpu.SemaphoreType.DMA] * 4,
        ],
        compiler_params=pltpu.CompilerParams(use_tc_tiling_on_sc=False),
    )
    def prop(src_hbm, dst_hbm, g_hbm, zeros_hbm, out_hbm,
             src_v, dst_v, buf, acc, gsems, ssems):
        c = lax.axis_index("c")
        s = lax.axis_index("s")
        wid = s * NC + c
        NB = 4
        pltpu.sync_copy(src_hbm.at[pl.ds(wid * RPW, RPW)], src_v)
        pltpu.sync_copy(dst_hbm.at[pl.ds(wid * RPW, RPW)], dst_v)

        pltpu.sync_copy(zeros_hbm.at[pl.ds(s * NPT, NPT)],
                        acc.at[pl.ds(s * NPT, NPT)])

        def fire_gather(j):
            return pltpu.async_copy(g_hbm.at[src_v.at[j]], buf.at[j % NB],
                                    gsems[j % NB])

        def fire_scatter(j):
            return pltpu.async_copy(buf.at[j % NB], acc.at[dst_v.at[j]],
                                    ssems[j % NB], add=True)

        gd = {0: fire_gather(0)}
        if RPW > 1:
            gd[1] = fire_gather(1)
        plsc.subcore_barrier()
        sd = {}
        waited = set()
        for j in range(RPW):
            gd[j].wait()
            sd[j] = fire_scatter(j)
            jj = j + 2
            if jj < RPW:
                if jj - NB >= 0:
                    sd[jj - NB].wait()
                    waited.add(jj - NB)
                gd[jj] = fire_gather(jj)
        for j in range(RPW):
            if j not in waited:
                sd[j].wait()
        plsc.subcore_barrier()
        pltpu.sync_copy(acc.at[pl.ds(s * NPT, NPT)],
                        out_hbm.at[c].at[pl.ds(s * NPT, NPT)])

    return prop


_prop_kernel = _make_prop_kernel(F)


NP8 = N // 8


def _tc_a1_body(x_ref, w1b_ref, h_ref):
    hP = jnp.dot(x_ref[:, 0, :], w1b_ref[0],
                 preferred_element_type=jnp.float32)
    for b in range(1, 8):
        hP = hP + jnp.dot(x_ref[:, b, :], w1b_ref[b],
                          preferred_element_type=jnp.float32)
    h_ref[...] = hP


def _tc_a2_body(d0_ref, d1_ref, s_ref, h_ref, dis_ref, g_ref):
    d = d0_ref[...] + d1_ref[...] + 1.0
    dis2 = lax.rsqrt(d)
    disP = jnp.dot(dis2, s_ref[...],
                   preferred_element_type=jnp.float32)
    dis_ref[...] = disP
    g_ref[...] = h_ref[...] * disP


def _tc_b_body(p_ref, g1_ref, dis_ref, b1p_ref, g2_ref):
    disP = dis_ref[...]
    z = (p_ref[0] + p_ref[1] + g1_ref[...]) * disP + b1p_ref[...][None, :]
    g2_ref[...] = jnp.maximum(z, 0.0) * disP


def _tc_c_body(p_ref, g2_ref, dis_ref, w2p_ref, b2p_ref, gsum_ref, out_ref):
    zP = (p_ref[0] + p_ref[1] + g2_ref[...]) * dis_ref[...]
    h2 = jnp.dot(zP, w2p_ref[...],
                 preferred_element_type=jnp.float32)
    h2 = h2 + b2p_ref[...][None, :]
    m = jnp.max(h2, axis=1, keepdims=True)
    e = jnp.exp(h2 - m)
    gs = jnp.dot(e, gsum_ref[...],
                 preferred_element_type=jnp.float32)
    out_ref[...] = h2 - m - jnp.log(gs)


_tc_a1 = pl.pallas_call(
    _tc_a1_body, out_shape=jax.ShapeDtypeStruct((NP8, 128), jnp.float32))
_tc_a2 = pl.pallas_call(
    _tc_a2_body,
    out_shape=(jax.ShapeDtypeStruct((NP8, 128), jnp.float32),
               jax.ShapeDtypeStruct((NP8, 128), jnp.float32)))
_tc_b = pl.pallas_call(
    _tc_b_body, out_shape=jax.ShapeDtypeStruct((NP8, 128), jnp.float32))
_tc_c = pl.pallas_call(
    _tc_c_body, out_shape=jax.ShapeDtypeStruct((NP8, 32), jnp.float32))


def kernel(x, edge_index, W1, b1, W2, b2):
    ei = edge_index.astype(jnp.int32)
    src2 = ei[0].reshape(ROWS, CH)
    dst2 = ei[1].reshape(ROWS, CH)
    ones_ch = jnp.ones((CH,), jnp.float32)
    zeros_n = jnp.zeros((N,), jnp.float32)
    zeros_nf = jnp.zeros((N, F), jnp.float32)

    eye8 = jnp.eye(8, dtype=jnp.float32)
    S = jnp.kron(eye8, jnp.ones((1, F), jnp.float32))
    W1B = jnp.stack([jnp.pad(W1, ((0, 0), (F * b, 128 - F * (b + 1))))
                     for b in range(8)])
    W2P = jnp.kron(eye8, W2)
    b1P = jnp.tile(b1, 8)
    b2P = jnp.tile(b2, 8)
    G = jnp.kron(eye8, jnp.ones((4, 4), jnp.float32))
    xp = x.reshape(NP8, 8, 128)

    deg0, deg1 = _deg_kernel(dst2, ones_ch, zeros_n)
    hP = _tc_a1(xp, W1B)
    d0 = deg0.reshape(NP8, 8)
    d1 = deg1.reshape(NP8, 8)
    disP, g1P = _tc_a2(d0, d1, S, hP)
    p1 = _prop_kernel(src2, dst2, g1P.reshape(N, F), zeros_nf)
    g2P = _tc_b(p1.reshape(2, NP8, 128), g1P, disP, b1P)
    p2 = _prop_kernel(src2, dst2, g2P.reshape(N, F), zeros_nf)
    outP = _tc_c(p2.reshape(2, NP8, 128), g2P, disP, W2P, b2P, G)
    return outP.reshape(N, 4)
